# baseline (speedup 1.0000x reference)
import functools
import math

import jax
import jax.numpy as jnp
from jax import lax
from jax.experimental import pallas as pl
from jax.experimental.pallas import tpu as pltpu
from jax.experimental.pallas import tpu_sc as plsc

_D = 64
_LANES = 16
_NC = 2
_NS = 16
_NW = _NC * _NS
_G = 256
_PITCH = _G + 1
_CHUNK = 128
_SCALE = math.sqrt(_D)


@functools.lru_cache(maxsize=None)
def _build(n_j: int, n_b: int):
    n_blk = n_b // _G
    n_tasks = n_j * n_blk
    assert n_tasks % _NW == 0
    tpw = n_tasks // _NW

    mesh = plsc.VectorSubcoreMesh(
        core_axis_name="c", subcore_axis_name="s",
        num_cores=_NC, num_subcores=_NS)

    @functools.partial(
        pl.kernel,
        out_type=jax.ShapeDtypeStruct((n_j, _D, n_b), jnp.float32),
        mesh=mesh,
        compiler_params=pltpu.CompilerParams(
            use_tc_tiling_on_sc=False, needs_layout_passes=False),
        scratch_types=[
            pltpu.VMEM((_G // _CHUNK, _CHUNK), jnp.int32),
            pltpu.VMEM((_G, _D), jnp.float32),
            pltpu.VMEM((_D, _PITCH), jnp.float32),
            pltpu.SemaphoreType.DMA,
        ],
    )
    def sc_embed(idx_hbm, table_hbm, out_hbm, idx_v, rows_v, tr_v, gsem):
        wid = lax.axis_index("s") * _NC + lax.axis_index("c")
        t0 = wid * tpw
        iot = lax.iota(jnp.int32, _LANES)
        dvecs = [iot + k * _LANES for k in range(_D // _LANES)]

        def task_body(t, carry):
            j = (t0 + t) // n_blk
            blk = (t0 + t) % n_blk
            pltpu.sync_copy(
                idx_hbm.at[j, pl.ds(blk * (_G // _CHUNK), _G // _CHUNK)],
                idx_v)
            cps = [
                pltpu.async_copy(
                    table_hbm.at[idx_v.at[c]],
                    rows_v.at[pl.ds(c * _CHUNK, _CHUNK)], gsem)
                for c in range(_G // _CHUNK)
            ]
            for cp in cps:
                cp.wait()

            def tr_body(b, carry2):
                col = jnp.full((_LANES,), 0, jnp.int32) + b
                for k in range(_D // _LANES):
                    v = rows_v[b, pl.ds(k * _LANES, _LANES)] * _SCALE
                    plsc.store_scatter(tr_v, [dvecs[k], col], v)
                return carry2

            lax.fori_loop(0, _G, tr_body, 0, unroll=8)

            pltpu.sync_copy(
                tr_v.at[:, pl.ds(0, _G)],
                out_hbm.at[j, :, pl.ds(blk * _G, _G)])
            return carry

        lax.fori_loop(0, tpw, task_body, 0)

    return sc_embed


def kernel(x, table):
    n_b, n_j = x.shape
    idx3 = x.T.reshape(n_j, n_b // _CHUNK, _CHUNK)
    out = _build(n_j, n_b)(idx3, table)
    return out.transpose(2, 0, 1)


# speedup vs baseline: 1.1747x; 1.1747x over previous
import functools
import math

import jax
import jax.numpy as jnp
from jax import lax
from jax.experimental import pallas as pl
from jax.experimental.pallas import tpu as pltpu
from jax.experimental.pallas import tpu_sc as plsc

_D = 64
_LANES = 16
_NC = 2
_NS = 16
_NW = _NC * _NS
_G = 256
_PITCH = _G + 1
_CHUNK = 128
_SCALE = math.sqrt(_D)


@functools.lru_cache(maxsize=None)
def _build(n_j: int, n_b: int):
    n_blk = n_b // _G
    n_tasks = n_j * n_blk
    assert n_tasks % _NW == 0
    tpw = n_tasks // _NW
    assert tpw % 2 == 0

    mesh = plsc.VectorSubcoreMesh(
        core_axis_name="c", subcore_axis_name="s",
        num_cores=_NC, num_subcores=_NS)

    @functools.partial(
        pl.kernel,
        out_type=jax.ShapeDtypeStruct((n_j, _D, n_b), jnp.float32),
        mesh=mesh,
        compiler_params=pltpu.CompilerParams(
            use_tc_tiling_on_sc=False, needs_layout_passes=False),
        scratch_types=[
            pltpu.VMEM((2, _G), jnp.int32),
            pltpu.VMEM((2, _G, _D), jnp.float32),
            pltpu.VMEM((2, _D, _PITCH), jnp.float32),
            pltpu.SemaphoreType.DMA((2,)),
            pltpu.SemaphoreType.DMA((2,)),
            pltpu.SemaphoreType.DMA((2,)),
        ],
    )
    def sc_embed(idx_hbm, table_hbm, out_hbm, idx_v, rows_v, tr_v,
                 isem, gsem, wsem):
        wid = lax.axis_index("s") * _NC + lax.axis_index("c")
        t0 = wid * tpw
        iot = lax.iota(jnp.int32, _LANES)
        dvecs = [iot + k * _LANES for k in range(_D // _LANES)]

        def idx_src(t):
            j = (t0 + t) // n_blk
            blk = (t0 + t) % n_blk
            return idx_hbm.at[j, pl.ds(blk * _G, _G)]

        def out_dst(t):
            j = (t0 + t) // n_blk
            blk = (t0 + t) % n_blk
            return out_hbm.at[j, :, pl.ds(blk * _G, _G)]

        def fire_idx(t, p):
            pltpu.async_copy(idx_src(t), idx_v.at[p], isem.at[p])

        def fire_gather(t, p):
            for c in range(_G // _CHUNK):
                pltpu.async_copy(
                    table_hbm.at[idx_v.at[p, pl.ds(c * _CHUNK, _CHUNK)]],
                    rows_v.at[p, pl.ds(c * _CHUNK, _CHUNK)], gsem.at[p])

        def wait_idx(t, p):
            pltpu.make_async_copy(idx_src(t), idx_v.at[p], isem.at[p]).wait()

        def wait_gather(t, p):
            for c in range(_G // _CHUNK):
                pltpu.make_async_copy(
                    table_hbm.at[idx_v.at[p, pl.ds(c * _CHUNK, _CHUNK)]],
                    rows_v.at[p, pl.ds(c * _CHUNK, _CHUNK)],
                    gsem.at[p]).wait()

        def fire_write(t, p):
            pltpu.async_copy(tr_v.at[p, :, pl.ds(0, _G)], out_dst(t),
                             wsem.at[p])

        def wait_write(t, p):
            pltpu.make_async_copy(tr_v.at[p, :, pl.ds(0, _G)], out_dst(t),
                                  wsem.at[p]).wait()

        fire_idx(0, 0)
        fire_idx(1, 1)
        wait_idx(0, 0)
        fire_gather(0, 0)

        def pair_body(t2, carry):
            for p in range(2):
                t = t2 * 2 + p
                q = 1 - p
                @pl.when(t + 1 < tpw)
                def _():
                    wait_idx(t + 1, q)
                    fire_gather(t + 1, q)

                wait_gather(t, p)

                @pl.when(t + 2 < tpw)
                def _():
                    fire_idx(t + 2, p)

                @pl.when(t >= 2)
                def _():
                    wait_write(t - 2, p)

                def tr_body(b, carry2):
                    col = jnp.full((_LANES,), 0, jnp.int32) + b
                    for k in range(_D // _LANES):
                        v = rows_v[p, b, pl.ds(k * _LANES, _LANES)] * _SCALE
                        plsc.store_scatter(tr_v.at[p], [dvecs[k], col], v)
                    return carry2

                lax.fori_loop(0, _G, tr_body, 0, unroll=8)
                fire_write(t, p)
            return carry

        lax.fori_loop(0, tpw // 2, pair_body, 0)
        wait_write(tpw - 2, 0)
        wait_write(tpw - 1, 1)

    return sc_embed


def kernel(x, table):
    n_b, n_j = x.shape
    out = _build(n_j, n_b)(x.T, table)
    return out.transpose(2, 0, 1)
